# Initial kernel scaffold; baseline (speedup 1.0000x reference)
import jax
import jax.numpy as jnp
from jax.experimental import pallas as pl


def kernel(coordinates, features):
    raise NotImplementedError("write your pallas kernel here")


# speedup vs baseline: 273.0317x; 273.0317x over previous
import functools

import jax
import jax.numpy as jnp
from jax.experimental import pallas as pl

K = 16
KP1 = K + 1


def _topk_kernel(coords_ref, idx_ref, *, rb: int, n: int):
    i = pl.program_id(1)
    c = coords_ref[0]
    rows = coords_ref[0, :, pl.ds(i * rb, rb)]
    col_sq = jnp.sum(c * c, axis=0, keepdims=True)
    row_sq = jnp.sum(rows * rows, axis=0)
    mm = jax.lax.dot_general(
        rows, c, (((0,), (0,)), ((), ())),
        preferred_element_type=jnp.float32,
    )
    dist = -2.0 * mm
    dist = dist + row_sq[:, None]
    dist = dist + col_sq
    dist = jnp.clip(dist, 1e-12, None)

    iota_n = jax.lax.broadcasted_iota(jnp.int32, (rb, n), 1)
    vals = dist
    for k in range(KP1):
        m = jnp.min(vals, axis=1, keepdims=True)
        am = jnp.min(jnp.where(vals == m, iota_n, n), axis=1, keepdims=True)
        idx_ref[0, k, :] = am[:, 0]
        if k + 1 < KP1:
            vals = jnp.where(iota_n == am, jnp.inf, vals)


def _gather_kernel(feat_ref, idx_ref, out_ref, *, nb: int, n: int, c: int):
    j = pl.program_id(1)
    f = feat_ref[0]
    fb = feat_ref[0, :, pl.ds(j * nb, nb)]
    nk = nb * K

    hp = jax.lax.Precision.HIGHEST
    i0 = jax.lax.broadcasted_iota(jnp.int32, (nb, nk), 0)
    i1 = jax.lax.broadcasted_iota(jnp.int32, (nb, nk), 1)
    expand = jnp.where(i0 == i1 // K, 1.0, 0.0).astype(jnp.float32)

    ids = idx_ref[0, 1:KP1, :].astype(jnp.float32)
    expanded = jax.lax.dot_general(
        ids, expand, (((1,), (0,)), ((), ())),
        preferred_element_type=jnp.float32, precision=hp)
    k0 = jax.lax.broadcasted_iota(jnp.int32, (K, nk), 0)
    k1 = jax.lax.broadcasted_iota(jnp.int32, (K, nk), 1)
    sel = jnp.where(k1 % K == k0, 1.0, 0.0)
    ids_flat = jnp.sum(expanded * sel, axis=0, keepdims=True)

    rep = jax.lax.dot_general(
        fb, expand, (((1,), (0,)), ((), ())),
        preferred_element_type=jnp.float32, precision=hp)

    mc = 512
    g = jnp.zeros((c, nk), jnp.float32)
    for s in range(n // mc):
        iota_m = jax.lax.broadcasted_iota(jnp.int32, (mc, nk), 0) + s * mc
        oh = jnp.where(iota_m.astype(jnp.float32) == ids_flat, 1.0, 0.0)
        g = g + jax.lax.dot_general(
            f[:, s * mc:(s + 1) * mc], oh, (((1,), (0,)), ((), ())),
            preferred_element_type=jnp.float32, precision=hp)

    out_ref[0, 0:c] = rep
    out_ref[0, c:2 * c] = g - rep


def kernel(coordinates, features):
    if features.ndim == 4 and features.shape[-1] == 1:
        features = jnp.squeeze(features, axis=-1)
    B, C, N = features.shape
    RB = 256
    NB = 128

    idx = pl.pallas_call(
        functools.partial(_topk_kernel, rb=RB, n=N),
        grid=(B, N // RB),
        in_specs=[pl.BlockSpec((1, 3, N), lambda b, i: (b, 0, 0))],
        out_specs=pl.BlockSpec((1, KP1, RB), lambda b, i: (b, 0, i)),
        out_shape=jax.ShapeDtypeStruct((B, KP1, N), jnp.int32),
    )(coordinates)

    out = pl.pallas_call(
        functools.partial(_gather_kernel, nb=NB, n=N, c=C),
        grid=(B, N // NB),
        in_specs=[
            pl.BlockSpec((1, C, N), lambda b, j: (b, 0, 0)),
            pl.BlockSpec((1, KP1, NB), lambda b, j: (b, 0, j)),
        ],
        out_specs=pl.BlockSpec((1, 2 * C, NB * K), lambda b, j: (b, 0, j)),
        out_shape=jax.ShapeDtypeStruct((B, 2 * C, N * K), jnp.float32),
    )(features, idx)
    return out.reshape(B, 2 * C, N, K)


# speedup vs baseline: 404.3413x; 1.4809x over previous
import functools

import jax
import jax.numpy as jnp
from jax.experimental import pallas as pl
from jax.experimental.pallas import tpu as pltpu

K = 16
KP1 = K + 1


def _topk_kernel(coords_ref, idx_ref, *, rb: int, n: int):
    i = pl.program_id(1)
    c = coords_ref[0]
    rows = coords_ref[0, :, pl.ds(i * rb, rb)]
    col_sq = jnp.sum(c * c, axis=0, keepdims=True)
    row_sq = jnp.sum(rows * rows, axis=0)
    mm = jax.lax.dot_general(
        rows, c, (((0,), (0,)), ((), ())),
        preferred_element_type=jnp.float32,
    )
    dist = -2.0 * mm
    dist = dist + row_sq[:, None]
    dist = dist + col_sq
    dist = jnp.clip(dist, 1e-12, None)

    iota_n = jax.lax.broadcasted_iota(jnp.int32, (rb, n), 1)
    vals = dist
    for k in range(KP1):
        m = jnp.min(vals, axis=1, keepdims=True)
        am = jnp.min(jnp.where(vals == m, iota_n, n), axis=1, keepdims=True)
        idx_ref[0, k, :] = am[:, 0]
        if k + 1 < KP1:
            vals = jnp.where(iota_n == am, jnp.inf, vals)


def _gather_kernel(feat_ref, idx_ref, out_ref, *, nb: int, n: int, c: int):
    j = pl.program_id(1)
    f = feat_ref[0]
    fb = feat_ref[0, :, pl.ds(j * nb, nb)]
    nk = nb * K

    hp = jax.lax.Precision.HIGHEST
    i0 = jax.lax.broadcasted_iota(jnp.int32, (nb, nk), 0)
    i1 = jax.lax.broadcasted_iota(jnp.int32, (nb, nk), 1)
    expand = jnp.where(i0 == i1 // K, 1.0, 0.0).astype(jnp.float32)

    ids = idx_ref[0, 1:KP1, :].astype(jnp.float32)
    expanded = jax.lax.dot_general(
        ids, expand, (((1,), (0,)), ((), ())),
        preferred_element_type=jnp.float32, precision=hp)
    k0 = jax.lax.broadcasted_iota(jnp.int32, (K, nk), 0)
    k1 = jax.lax.broadcasted_iota(jnp.int32, (K, nk), 1)
    sel = jnp.where(k1 % K == k0, 1.0, 0.0)
    ids_flat = jnp.sum(expanded * sel, axis=0, keepdims=True)

    rep = jax.lax.dot_general(
        fb, expand, (((1,), (0,)), ((), ())),
        preferred_element_type=jnp.float32, precision=hp)

    f_hi = f.astype(jnp.bfloat16)
    f_lo = (f - f_hi.astype(jnp.float32)).astype(jnp.bfloat16)
    mc = 512
    g = jnp.zeros((c, nk), jnp.float32)
    for s in range(n // mc):
        iota_m = jax.lax.broadcasted_iota(jnp.int32, (mc, nk), 0) + s * mc
        oh = jnp.where(
            iota_m.astype(jnp.float32) == ids_flat, 1.0, 0.0
        ).astype(jnp.bfloat16)
        g = g + jax.lax.dot_general(
            f_hi[:, s * mc:(s + 1) * mc], oh, (((1,), (0,)), ((), ())),
            preferred_element_type=jnp.float32)
        g = g + jax.lax.dot_general(
            f_lo[:, s * mc:(s + 1) * mc], oh, (((1,), (0,)), ((), ())),
            preferred_element_type=jnp.float32)

    out_ref[0, 0:c] = rep
    out_ref[0, c:2 * c] = g - rep


def kernel(coordinates, features):
    if features.ndim == 4 and features.shape[-1] == 1:
        features = jnp.squeeze(features, axis=-1)
    B, C, N = features.shape
    RB = 256
    NB = 128

    idx = pl.pallas_call(
        functools.partial(_topk_kernel, rb=RB, n=N),
        grid=(B, N // RB),
        in_specs=[pl.BlockSpec((1, 3, N), lambda b, i: (b, 0, 0))],
        out_specs=pl.BlockSpec((1, KP1, RB), lambda b, i: (b, 0, i)),
        out_shape=jax.ShapeDtypeStruct((B, KP1, N), jnp.int32),
        compiler_params=pltpu.CompilerParams(
            dimension_semantics=("parallel", "parallel")),
    )(coordinates)

    out = pl.pallas_call(
        functools.partial(_gather_kernel, nb=NB, n=N, c=C),
        grid=(B, N // NB),
        in_specs=[
            pl.BlockSpec((1, C, N), lambda b, j: (b, 0, 0)),
            pl.BlockSpec((1, KP1, NB), lambda b, j: (b, 0, j)),
        ],
        out_specs=pl.BlockSpec((1, 2 * C, NB * K), lambda b, j: (b, 0, j)),
        out_shape=jax.ShapeDtypeStruct((B, 2 * C, N * K), jnp.float32),
        compiler_params=pltpu.CompilerParams(
            dimension_semantics=("parallel", "parallel")),
    )(features, idx)
    return out.reshape(B, 2 * C, N, K)


# speedup vs baseline: 457.9772x; 1.1327x over previous
import functools

import jax
import jax.numpy as jnp
from jax.experimental import pallas as pl
from jax.experimental.pallas import tpu as pltpu

K = 16
KP1 = K + 1


def _topk_kernel(coords_ref, idx_ref, *, rb: int, n: int):
    i = pl.program_id(1)
    c = coords_ref[0]
    rows = coords_ref[0, :, pl.ds(i * rb, rb)]
    col_sq = jnp.sum(c * c, axis=0, keepdims=True)
    row_sq = jnp.sum(rows * rows, axis=0)
    mm = jax.lax.dot_general(
        rows, c, (((0,), (0,)), ((), ())),
        preferred_element_type=jnp.float32,
    )
    dist = -2.0 * mm
    dist = dist + row_sq[:, None]
    dist = dist + col_sq
    dist = jnp.clip(dist, 1e-12, None)

    iota_n = jax.lax.broadcasted_iota(jnp.int32, (rb, n), 1)
    vals = dist
    for k in range(KP1):
        m = jnp.min(vals, axis=1, keepdims=True)
        am = jnp.min(jnp.where(vals == m, iota_n, n), axis=1, keepdims=True)
        idx_ref[0, k, :] = am[:, 0]
        if k + 1 < KP1:
            vals = jnp.where(iota_n == am, jnp.inf, vals)


def _gather_kernel(feat_ref, idx_ref, out_ref, *, nb: int, n: int, c: int):
    j = pl.program_id(1)
    f = feat_ref[0]
    fb = feat_ref[0, :, pl.ds(j * nb, nb)]

    f_hi = f.astype(jnp.bfloat16)
    f_lo = (f - f_hi.astype(jnp.float32)).astype(jnp.bfloat16)
    mc = 512
    for k in range(K):
        ids_k = idx_ref[0, k + 1:k + 2, :]
        g = jnp.zeros((c, nb), jnp.float32)
        for s in range(n // mc):
            iota_m = jax.lax.broadcasted_iota(jnp.int32, (mc, nb), 0) + s * mc
            oh = jnp.where(iota_m == ids_k, 1.0, 0.0).astype(jnp.bfloat16)
            g = g + jax.lax.dot_general(
                f_hi[:, s * mc:(s + 1) * mc], oh, (((1,), (0,)), ((), ())),
                preferred_element_type=jnp.float32)
            g = g + jax.lax.dot_general(
                f_lo[:, s * mc:(s + 1) * mc], oh, (((1,), (0,)), ((), ())),
                preferred_element_type=jnp.float32)
        out_ref[0, 0:c, k, :] = fb
        out_ref[0, c:2 * c, k, :] = g - fb


def kernel(coordinates, features):
    if features.ndim == 4 and features.shape[-1] == 1:
        features = jnp.squeeze(features, axis=-1)
    B, C, N = features.shape
    RB = 256
    NB = 128

    idx = pl.pallas_call(
        functools.partial(_topk_kernel, rb=RB, n=N),
        grid=(B, N // RB),
        in_specs=[pl.BlockSpec((1, 3, N), lambda b, i: (b, 0, 0))],
        out_specs=pl.BlockSpec((1, KP1, RB), lambda b, i: (b, 0, i)),
        out_shape=jax.ShapeDtypeStruct((B, KP1, N), jnp.int32),
        compiler_params=pltpu.CompilerParams(
            dimension_semantics=("parallel", "parallel")),
    )(coordinates)

    out = pl.pallas_call(
        functools.partial(_gather_kernel, nb=NB, n=N, c=C),
        grid=(B, N // NB),
        in_specs=[
            pl.BlockSpec((1, C, N), lambda b, j: (b, 0, 0)),
            pl.BlockSpec((1, KP1, NB), lambda b, j: (b, 0, j)),
        ],
        out_specs=pl.BlockSpec((1, 2 * C, K, NB), lambda b, j: (b, 0, 0, j)),
        out_shape=jax.ShapeDtypeStruct((B, 2 * C, K, N), jnp.float32),
        compiler_params=pltpu.CompilerParams(
            dimension_semantics=("parallel", "parallel")),
    )(features, idx)
    return out.transpose(0, 1, 3, 2)


# speedup vs baseline: 604.8102x; 1.3206x over previous
import functools

import jax
import jax.numpy as jnp
from jax.experimental import pallas as pl
from jax.experimental.pallas import tpu as pltpu

K = 16
KP1 = K + 1


def _topk_kernel(coords_ref, idx_ref, *, rb: int, n: int):
    i = pl.program_id(1)
    c = coords_ref[0]
    rows = coords_ref[0, :, pl.ds(i * rb, rb)]
    col_sq = jnp.sum(c * c, axis=0, keepdims=True)
    row_sq = jnp.sum(rows * rows, axis=0)
    mm = jax.lax.dot_general(
        rows, c, (((0,), (0,)), ((), ())),
        preferred_element_type=jnp.float32,
    )
    dist = -2.0 * mm
    dist = dist + row_sq[:, None]
    dist = dist + col_sq
    dist = jnp.clip(dist, 1e-12, None)

    iota_n = jax.lax.broadcasted_iota(jnp.int32, (rb, n), 1)
    vals = dist
    for k in range(KP1):
        m = jnp.min(vals, axis=1, keepdims=True)
        am = jnp.min(jnp.where(vals == m, iota_n, n), axis=1, keepdims=True)
        idx_ref[0, k, :] = am[:, 0]
        if k + 1 < KP1:
            vals = jnp.where(iota_n == am, jnp.inf, vals)


def _gather_kernel(feat_ref, idx_ref, out_ref, *, nb: int, n: int, c: int):
    j = pl.program_id(1)
    f = feat_ref[0]
    fb = feat_ref[0, :, pl.ds(j * nb, nb)]

    f_hi = f.astype(jnp.bfloat16)
    f_lo = (f - f_hi.astype(jnp.float32)).astype(jnp.bfloat16)
    mc = 512
    for k in range(K):
        ids_k = idx_ref[0, k + 1:k + 2, :]
        g = jnp.zeros((c, nb), jnp.float32)
        for s in range(n // mc):
            iota_m = jax.lax.broadcasted_iota(jnp.int32, (mc, nb), 0) + s * mc
            oh = jnp.where(iota_m == ids_k, 1.0, 0.0).astype(jnp.bfloat16)
            g = g + jax.lax.dot_general(
                f_hi[:, s * mc:(s + 1) * mc], oh, (((1,), (0,)), ((), ())),
                preferred_element_type=jnp.float32)
            g = g + jax.lax.dot_general(
                f_lo[:, s * mc:(s + 1) * mc], oh, (((1,), (0,)), ((), ())),
                preferred_element_type=jnp.float32)
        out_ref[0, 0:c, k, :] = fb
        out_ref[0, c:2 * c, k, :] = g - fb


def kernel(coordinates, features):
    if features.ndim == 4 and features.shape[-1] == 1:
        features = jnp.squeeze(features, axis=-1)
    B, C, N = features.shape
    RB = 256
    NB = 512

    idx = pl.pallas_call(
        functools.partial(_topk_kernel, rb=RB, n=N),
        grid=(B, N // RB),
        in_specs=[pl.BlockSpec((1, 3, N), lambda b, i: (b, 0, 0))],
        out_specs=pl.BlockSpec((1, KP1, RB), lambda b, i: (b, 0, i)),
        out_shape=jax.ShapeDtypeStruct((B, KP1, N), jnp.int32),
        compiler_params=pltpu.CompilerParams(
            dimension_semantics=("parallel", "parallel")),
    )(coordinates)

    out = pl.pallas_call(
        functools.partial(_gather_kernel, nb=NB, n=N, c=C),
        grid=(B, N // NB),
        in_specs=[
            pl.BlockSpec((1, C, N), lambda b, j: (b, 0, 0)),
            pl.BlockSpec((1, KP1, NB), lambda b, j: (b, 0, j)),
        ],
        out_specs=pl.BlockSpec((1, 2 * C, K, NB), lambda b, j: (b, 0, 0, j)),
        out_shape=jax.ShapeDtypeStruct((B, 2 * C, K, N), jnp.float32),
        compiler_params=pltpu.CompilerParams(
            dimension_semantics=("parallel", "parallel")),
    )(features, idx)
    return out.transpose(0, 1, 3, 2)
